# Initial kernel scaffold; baseline (speedup 1.0000x reference)
import functools

import jax
import jax.numpy as jnp
from jax import lax
from jax.experimental import pallas as pl
from jax.experimental.pallas import tpu as pltpu
from jax.experimental.pallas import tpu_sc as plsc

N_NODES_C = 10000
N_EDGES_C = 320000
D_C = 128

NC = 2
NS = 16
NW = NC * NS

CHUNK = 100
E_PER_TILE = N_EDGES_C // NS
N_CH1 = E_PER_TILE // CHUNK
E_PER_W = N_EDGES_C // NW
N_CH2 = E_PER_W // CHUNK
ROWS_IDX = N_EDGES_C // CHUNK
POOL_PER_TILE = N_NODES_C // NS


def _body(edges_hbm, ei0_hbm, ei1_hbm, ep_hbm, z_hbm, out_hbm,
          pool, idx0_v, idx1_v, idxp_v, ebuf, pbuf):
    c = lax.axis_index("c")
    s = lax.axis_index("s")
    w = s * NC + c

    pltpu.sync_copy(z_hbm.at[pl.ds(s * POOL_PER_TILE, POOL_PER_TILE)],
                    pool.at[pl.ds(s * POOL_PER_TILE, POOL_PER_TILE)])
    plsc.subcore_barrier()

    pltpu.sync_copy(ei0_hbm.at[pl.ds(s * N_CH1, N_CH1)], idx0_v)

    def p1(j, _):
        base = s * E_PER_TILE + j * CHUNK
        pltpu.sync_copy(edges_hbm.at[pl.ds(base, CHUNK)], ebuf)
        pltpu.sync_copy(ebuf, pool.at[idx0_v.at[j]], add=True)
        return 0

    lax.fori_loop(0, N_CH1, p1, 0)
    plsc.subcore_barrier()

    pltpu.sync_copy(ei1_hbm.at[pl.ds(w * N_CH2, N_CH2)], idx1_v)
    pltpu.sync_copy(ep_hbm.at[pl.ds(w * N_CH2, N_CH2)], idxp_v)

    def p2(j, _):
        pltpu.sync_copy(pool.at[idx1_v.at[j]], pbuf)
        pltpu.sync_copy(edges_hbm.at[idxp_v.at[j]], ebuf)

        def sub_row(r, _):
            for cc in range(D_C // 16):
                sl = pl.ds(cc * 16, 16)
                pbuf[r, sl] = pbuf[r, sl] - ebuf[r, sl]
            return 0

        lax.fori_loop(0, CHUNK, sub_row, 0)
        pltpu.sync_copy(pbuf, out_hbm.at[pl.ds(w * E_PER_W + j * CHUNK, CHUNK)])
        return 0

    lax.fori_loop(0, N_CH2, p2, 0)


@jax.jit
def _run(edges, ei0, ei1, ep, z):
    mesh = plsc.VectorSubcoreMesh(core_axis_name="c", subcore_axis_name="s")
    f = pl.kernel(
        _body,
        out_type=jax.ShapeDtypeStruct((N_EDGES_C, D_C), jnp.float32),
        mesh=mesh,
        scratch_types=[
            pltpu.VMEM_SHARED((N_NODES_C, D_C), jnp.float32),
            pltpu.VMEM((N_CH1, CHUNK), jnp.int32),
            pltpu.VMEM((N_CH2, CHUNK), jnp.int32),
            pltpu.VMEM((N_CH2, CHUNK), jnp.int32),
            pltpu.VMEM((CHUNK, D_C), jnp.float32),
            pltpu.VMEM((CHUNK, D_C), jnp.float32),
        ],
    )
    return f(edges, ei0, ei1, ep, z)


def kernel(nodes, edges, edge_index, edge_pair):
    ei0 = edge_index[0].astype(jnp.int32).reshape(ROWS_IDX, CHUNK)
    ei1 = edge_index[1].astype(jnp.int32).reshape(ROWS_IDX, CHUNK)
    ep = edge_pair[0].astype(jnp.int32).reshape(ROWS_IDX, CHUNK)
    z = jnp.zeros((N_NODES_C, D_C), jnp.float32)
    return _run(edges, ei0, ei1, ep, z)


# speedup vs baseline: 3.0930x; 3.0930x over previous
import jax
import jax.numpy as jnp
from jax import lax
from jax.experimental import pallas as pl
from jax.experimental.pallas import tpu as pltpu
from jax.experimental.pallas import tpu_sc as plsc

N_NODES_C = 10000
N_EDGES_C = 320000
D_C = 128

NC = 2
NS = 16
NW = NC * NS

CHUNK = 80
E_PER_W = N_EDGES_C // NW
N_CH = E_PER_W // CHUNK
ZCHUNK = 640
NZ = ZCHUNK // CHUNK


def _tile_rows(s):
    return pl.multiple_of(s * ZCHUNK - (s // (NS - 1)) * 240, 8)


def _body_pool(edges_hbm, ei0_hbm, z_hbm, part_hbm, pool, idx_v, ebuf):
    c = lax.axis_index("c")
    s = lax.axis_index("s")
    w = s * NC + c

    zoff = _tile_rows(s)
    pltpu.sync_copy(z_hbm.at[pl.ds(zoff, ZCHUNK)], pool.at[pl.ds(zoff, ZCHUNK)])
    plsc.subcore_barrier()

    pltpu.sync_copy(ei0_hbm.at[w], idx_v)

    def p1(j, _):
        pltpu.sync_copy(edges_hbm.at[pl.ds(w * E_PER_W + j * CHUNK, CHUNK)],
                        ebuf)
        pltpu.sync_copy(ebuf, pool.at[idx_v.at[j]], add=True)
        return 0

    lax.fori_loop(0, N_CH, p1, 0)
    plsc.subcore_barrier()

    pltpu.sync_copy(pool.at[pl.ds(zoff, ZCHUNK)],
                    part_hbm.at[c].at[pl.ds(zoff, ZCHUNK)])


def _body_out(part_hbm, edges_hbm, ei1_hbm, ep_hbm, out_hbm,
              pool, idx1_v, idxp_v, ebuf, pbuf):
    c = lax.axis_index("c")
    s = lax.axis_index("s")
    w = s * NC + c

    zoff = _tile_rows(s)

    def comb(k, _):
        r0 = pl.multiple_of(zoff + k * CHUNK, 8)
        pltpu.sync_copy(part_hbm.at[0].at[pl.ds(r0, CHUNK)], ebuf)
        pltpu.sync_copy(part_hbm.at[1].at[pl.ds(r0, CHUNK)], pbuf)

        def add_row(r, _):
            for cc in range(D_C // 16):
                sl = pl.ds(cc * 16, 16)
                pbuf[r, sl] = pbuf[r, sl] + ebuf[r, sl]
            return 0

        lax.fori_loop(0, CHUNK, add_row, 0)
        pltpu.sync_copy(pbuf, pool.at[pl.ds(r0, CHUNK)])
        return 0

    lax.fori_loop(0, NZ, comb, 0)
    plsc.subcore_barrier()

    pltpu.sync_copy(ei1_hbm.at[w, 0], idx1_v)
    pltpu.sync_copy(ep_hbm.at[w, 0], idxp_v)

    def p2(j, _):
        pltpu.sync_copy(pool.at[idx1_v.at[pl.ds(j * CHUNK, CHUNK)]], pbuf)
        pltpu.sync_copy(edges_hbm.at[idxp_v.at[pl.ds(j * CHUNK, CHUNK)]],
                        ebuf)

        def sub_row(r, _):
            for cc in range(D_C // 16):
                sl = pl.ds(cc * 16, 16)
                pbuf[r, sl] = pbuf[r, sl] - ebuf[r, sl]
            return 0

        lax.fori_loop(0, CHUNK, sub_row, 0)
        pltpu.sync_copy(pbuf, out_hbm.at[pl.ds(w * E_PER_W + j * CHUNK, CHUNK)])
        return 0

    lax.fori_loop(0, N_CH, p2, 0)


@jax.jit
def _run(edges, ei0, ei1, ep, z):
    mesh = plsc.VectorSubcoreMesh(core_axis_name="c", subcore_axis_name="s")
    part = pl.kernel(
        _body_pool,
        out_type=jax.ShapeDtypeStruct((NC, N_NODES_C, D_C), jnp.float32),
        mesh=mesh,
        scratch_types=[
            pltpu.VMEM_SHARED((N_NODES_C, D_C), jnp.float32),
            pltpu.VMEM((N_CH, CHUNK), jnp.int32),
            pltpu.VMEM((CHUNK, D_C), jnp.float32),
        ],
    )(edges, ei0, z)
    out = pl.kernel(
        _body_out,
        out_type=jax.ShapeDtypeStruct((N_EDGES_C, D_C), jnp.float32),
        mesh=mesh,
        scratch_types=[
            pltpu.VMEM_SHARED((N_NODES_C, D_C), jnp.float32),
            pltpu.VMEM((E_PER_W,), jnp.int32),
            pltpu.VMEM((E_PER_W,), jnp.int32),
            pltpu.VMEM((CHUNK, D_C), jnp.float32),
            pltpu.VMEM((CHUNK, D_C), jnp.float32),
        ],
    )(part, edges, ei1, ep)
    return out


def kernel(nodes, edges, edge_index, edge_pair):
    ei0 = edge_index[0].astype(jnp.int32).reshape(NW, N_CH, CHUNK)
    ei1 = edge_index[1].astype(jnp.int32).reshape(NW, 1, E_PER_W)
    ep = edge_pair[0].astype(jnp.int32).reshape(NW, 1, E_PER_W)
    z = jnp.zeros((N_NODES_C, D_C), jnp.float32)
    return _run(edges, ei0, ei1, ep, z)
